# baseline (speedup 1.0000x reference)
import functools

import jax
import jax.numpy as jnp
from jax import lax
from jax.experimental import pallas as pl
from jax.experimental.pallas import tpu as pltpu
from jax.experimental.pallas import tpu_sc as plsc


def _broadcast_emb(B, S, D, dtype):
    info = plsc.get_sparse_core_info()
    nw = info.num_cores * info.num_subcores
    rows_per_w = S // nw
    mesh = plsc.VectorSubcoreMesh(core_axis_name="c", subcore_axis_name="s")

    @functools.partial(
        pl.kernel,
        mesh=mesh,
        out_type=jax.ShapeDtypeStruct((B, S, D), dtype),
        scratch_types=[pltpu.SemaphoreType.DMA],
    )
    def k(emb_hbm, out_hbm, sem):
        wid = lax.axis_index("s") * info.num_cores + lax.axis_index("c")
        base = wid * rows_per_w
        copies = [
            pltpu.async_copy(
                emb_hbm.at[pl.ds(base, rows_per_w), :],
                out_hbm.at[b, pl.ds(base, rows_per_w), :],
                sem,
            )
            for b in range(B)
        ]
        for c in copies:
            c.wait()

    return k


def kernel(x, emb):
    B, S, D = x.shape
    return _broadcast_emb(B, S, D, emb.dtype)(emb)


# speedup vs baseline: 55.2776x; 55.2776x over previous
import functools

import jax
import jax.numpy as jnp
from jax import lax
from jax.experimental import pallas as pl
from jax.experimental.pallas import tpu as pltpu
from jax.experimental.pallas import tpu_sc as plsc


def _broadcast_emb(B, S, D, dtype):
    info = plsc.get_sparse_core_info()
    nw = info.num_cores * info.num_subcores
    rows_per_w = S // nw
    chunk = 64
    n_chunks = rows_per_w // chunk
    mesh = plsc.VectorSubcoreMesh(core_axis_name="c", subcore_axis_name="s")

    @functools.partial(
        pl.kernel,
        mesh=mesh,
        out_type=jax.ShapeDtypeStruct((B, S, D), dtype),
        scratch_types=[
            pltpu.VMEM((chunk, D), dtype),
            pltpu.SemaphoreType.DMA,
        ],
    )
    def k(emb_hbm, out_hbm, buf, sem):
        wid = lax.axis_index("s") * info.num_cores + lax.axis_index("c")
        base = wid * rows_per_w
        for i in range(n_chunks):
            r0 = base + i * chunk
            pltpu.sync_copy(emb_hbm.at[pl.ds(r0, chunk), :], buf)
            copies = [
                pltpu.async_copy(buf, out_hbm.at[b, pl.ds(r0, chunk), :], sem)
                for b in range(B)
            ]
            for c in copies:
                c.wait()

    return k


def kernel(x, emb):
    B, S, D = x.shape
    return _broadcast_emb(B, S, D, emb.dtype)(emb)


# speedup vs baseline: 55.8839x; 1.0110x over previous
import functools

import jax
import jax.numpy as jnp
from jax import lax
from jax.experimental import pallas as pl
from jax.experimental.pallas import tpu as pltpu
from jax.experimental.pallas import tpu_sc as plsc


def _broadcast_emb(B, S, D, dtype):
    info = plsc.get_sparse_core_info()
    nw = info.num_cores * info.num_subcores
    rows_per_w = S // nw
    sizes = [64, 56, 64, 56, 16]
    offs = [0, 64, 120, 184, 240]
    assert sum(sizes) == rows_per_w
    mesh = plsc.VectorSubcoreMesh(core_axis_name="c", subcore_axis_name="s")

    @functools.partial(
        pl.kernel,
        mesh=mesh,
        out_type=jax.ShapeDtypeStruct((B, S, D), dtype),
        scratch_types=[
            pltpu.VMEM((64, D), dtype),
            pltpu.VMEM((56, D), dtype),
            pltpu.SemaphoreType.DMA,
            pltpu.SemaphoreType.DMA,
            pltpu.SemaphoreType.DMA,
            pltpu.SemaphoreType.DMA,
        ],
    )
    def k(emb_hbm, out_hbm, buf0, buf1, rsem0, rsem1, wsem0, wsem1):
        wid = lax.axis_index("s") * info.num_cores + lax.axis_index("c")
        base = wid * rows_per_w
        bufs = (buf0, buf1)
        rsems = (rsem0, rsem1)
        wsems = (wsem0, wsem1)
        n = len(sizes)

        def start_read(i):
            c = i % 2
            return pltpu.async_copy(
                emb_hbm.at[pl.ds(base + offs[i], sizes[i]), :],
                bufs[c].at[pl.ds(0, sizes[i]), :],
                rsems[c],
            )

        reads = {0: start_read(0)}
        writes = {}
        for i in range(n):
            c = i % 2
            reads.pop(i).wait()
            if i + 1 < n:
                for w in writes.pop(i - 1, ()):
                    w.wait()
                reads[i + 1] = start_read(i + 1)
            r0 = base + offs[i]
            writes[i] = [
                pltpu.async_copy(
                    bufs[c].at[pl.ds(0, sizes[i]), :],
                    out_hbm.at[b, pl.ds(r0, sizes[i]), :],
                    wsems[c],
                )
                for b in range(B)
            ]
        for ws in writes.values():
            for w in ws:
                w.wait()

    return k


def kernel(x, emb):
    B, S, D = x.shape
    return _broadcast_emb(B, S, D, emb.dtype)(emb)


# speedup vs baseline: 56.4311x; 1.0098x over previous
import functools

import jax
import jax.numpy as jnp
from jax import lax
from jax.experimental import pallas as pl
from jax.experimental.pallas import tpu as pltpu
from jax.experimental.pallas import tpu_sc as plsc


def _broadcast_emb(B, S, D, dtype):
    info = plsc.get_sparse_core_info()
    nw = info.num_cores * info.num_subcores
    rows_per_w = S // nw
    sizes = [120, 120, 16]
    offs = [0, 120, 240]
    assert sum(sizes) == rows_per_w
    mesh = plsc.VectorSubcoreMesh(core_axis_name="c", subcore_axis_name="s")

    @functools.partial(
        pl.kernel,
        mesh=mesh,
        out_type=jax.ShapeDtypeStruct((B, S, D), dtype),
        scratch_types=[
            pltpu.VMEM((120, D), dtype),
            pltpu.SemaphoreType.DMA,
        ],
    )
    def k(emb_hbm, out_hbm, buf, sem):
        wid = lax.axis_index("s") * info.num_cores + lax.axis_index("c")
        base = wid * rows_per_w
        for i in range(len(sizes)):
            r0 = base + offs[i]
            pltpu.sync_copy(
                emb_hbm.at[pl.ds(r0, sizes[i]), :],
                buf.at[pl.ds(0, sizes[i]), :],
            )
            copies = [
                pltpu.async_copy(
                    buf.at[pl.ds(0, sizes[i]), :],
                    out_hbm.at[b, pl.ds(r0, sizes[i]), :],
                    sem,
                )
                for b in range(B)
            ]
            for c in copies:
                c.wait()

    return k


def kernel(x, emb):
    B, S, D = x.shape
    return _broadcast_emb(B, S, D, emb.dtype)(emb)
